# baseline (speedup 1.0000x reference)
import functools

import jax
import jax.numpy as jnp
from jax import lax
from jax.experimental import pallas as pl
from jax.experimental.pallas import tpu as pltpu
from jax.experimental.pallas import tpu_sc as plsc

N = 10000
E = 320000
NCORE = 2
NSUB = 16
NW = NCORE * NSUB
B = 128
NB = 79
NBS = NB + 3
EPT = NB * B
EPAD = EPT * NW
NPAD = 10240
CP = 128
RB = 2000
NRB = N // RB

_mesh = functools.partial(
    plsc.VectorSubcoreMesh, core_axis_name="c", subcore_axis_name="s"
)


def _sc_scatter(nc, srcs, dst4, gflat, zeros_c, gather=True):
    def body(srcs_r, dst4_r, g_r, z_r, p_r, srcv, rows0, rows1, dstb0, dstb1,
             acc, semg0, semg1, semd0, semd1):
        core = lax.axis_index("c")
        sub = lax.axis_index("s")
        w = core * NSUB + sub
        if not gather:
            pltpu.sync_copy(g_r.at[pl.ds(0, B)], rows0)
            pltpu.sync_copy(g_r.at[pl.ds(0, B)], rows1)
        for c in range(nc):
            pltpu.sync_copy(srcs_r.at[c, w], srcv)
            pltpu.sync_copy(z_r, acc.at[pl.ds(sub * 640, 640)])
            plsc.subcore_barrier()

            if gather:
                pltpu.async_copy(g_r.at[srcv.at[0]], rows0, semg0)
                pltpu.async_copy(g_r.at[srcv.at[1]], rows1, semg1)
            pltpu.async_copy(dst4_r.at[w, 0], dstb0, semd0)
            pltpu.async_copy(dst4_r.at[w, 1], dstb1, semd1)

            @pl.loop(0, NB, step=2)
            def _(j):
                if gather:
                    pltpu.make_async_copy(g_r.at[srcv.at[j]], rows0,
                                          semg0).wait()
                pltpu.make_async_copy(dst4_r.at[w, j], dstb0, semd0).wait()
                pltpu.sync_copy(rows0, acc.at[dstb0], add=True)
                if gather:
                    pltpu.async_copy(g_r.at[srcv.at[j + 2]], rows0, semg0)
                pltpu.async_copy(dst4_r.at[w, j + 2], dstb0, semd0)
                if gather:
                    pltpu.make_async_copy(g_r.at[srcv.at[j + 1]], rows1,
                                          semg1).wait()
                pltpu.make_async_copy(dst4_r.at[w, j + 1], dstb1, semd1).wait()
                pltpu.sync_copy(rows1, acc.at[dstb1], add=True)
                if gather:
                    pltpu.async_copy(g_r.at[srcv.at[j + 3]], rows1, semg1)
                pltpu.async_copy(dst4_r.at[w, j + 3], dstb1, semd1)

            if gather:
                pltpu.make_async_copy(g_r.at[srcv.at[0]], rows0, semg0).wait()
                pltpu.make_async_copy(g_r.at[srcv.at[0]], rows1, semg1).wait()
            pltpu.make_async_copy(dst4_r.at[w, 0], dstb0, semd0).wait()
            pltpu.make_async_copy(dst4_r.at[w, 1], dstb1, semd1).wait()
            plsc.subcore_barrier()
            pltpu.sync_copy(
                acc.at[pl.ds(sub * 624, 624)],
                p_r.at[c, core, pl.ds(sub * 624, 624)],
            )

            @pl.when(sub == 15)
            def _():
                pltpu.sync_copy(
                    acc.at[pl.ds(9984, 16)],
                    p_r.at[c, core, pl.ds(9984, 16)],
                )

            plsc.subcore_barrier()

    return pl.kernel(
        body,
        out_type=jax.ShapeDtypeStruct((nc, NCORE, N, CP), jnp.float32),
        mesh=_mesh(),
        scratch_types=[
            pltpu.VMEM((NBS, B), jnp.int32),
            pltpu.VMEM((B, CP), jnp.float32),
            pltpu.VMEM((B, CP), jnp.float32),
            pltpu.VMEM((B,), jnp.int32),
            pltpu.VMEM((B,), jnp.int32),
            pltpu.VMEM_SHARED((NPAD, CP), jnp.float32),
            pltpu.SemaphoreType.DMA,
            pltpu.SemaphoreType.DMA,
            pltpu.SemaphoreType.DMA,
            pltpu.SemaphoreType.DMA,
        ],
    )(srcs, dst4, gflat, zeros_c)


def _tc_dis(degp):
    def body(d_ref, o_ref):
        o_ref[...] = lax.rsqrt(d_ref[0] + d_ref[1] + 1.0)

    return pl.pallas_call(
        body,
        out_shape=jax.ShapeDtypeStruct((N, CP), jnp.float32),
    )(degp)


def _tc_matmul(h3, W3, dis):
    nci, _, _ = h3.shape
    nco, din_pad, _ = W3.shape

    def body(h_ref, w_ref, dis_ref, o_ref):
        acc = jnp.zeros((RB, CP), jnp.float32)
        for i in range(nci):
            acc += jnp.dot(
                h_ref[i],
                w_ref[0, i * CP : (i + 1) * CP, :],
                preferred_element_type=jnp.float32,
            )
        o_ref[0] = dis_ref[...] * acc

    return pl.pallas_call(
        body,
        grid=(NRB, nco),
        in_specs=[
            pl.BlockSpec((nci, RB, CP), lambda r, c: (0, r, 0)),
            pl.BlockSpec((1, din_pad, CP), lambda r, c: (c, 0, 0)),
            pl.BlockSpec((RB, 1), lambda r, c: (r, 0)),
        ],
        out_specs=pl.BlockSpec((1, RB, CP), lambda r, c: (c, r, 0)),
        out_shape=jax.ShapeDtypeStruct((nco, N, CP), jnp.float32),
    )(h3, W3, dis)


def _tc_combine(P, g3, dis, b3, skip3):
    nc = P.shape[0]
    has_skip = skip3 is not None

    def body(p_ref, g_ref, dis_ref, b_ref, *rest):
        if has_skip:
            s_ref, o_ref = rest
        else:
            (o_ref,) = rest
        v = p_ref[0, 0] + p_ref[0, 1] + g_ref[0]
        v = dis_ref[...] * v + b_ref[0]
        if has_skip:
            v += s_ref[0]
        o_ref[0] = jnp.maximum(v, 0.0)

    in_specs = [
        pl.BlockSpec((1, 2, RB, CP), lambda c, r: (c, 0, r, 0)),
        pl.BlockSpec((1, RB, CP), lambda c, r: (c, r, 0)),
        pl.BlockSpec((RB, 1), lambda c, r: (r, 0)),
        pl.BlockSpec((1, 1, CP), lambda c, r: (c, 0, 0)),
    ]
    args = [P, g3, dis, b3]
    if has_skip:
        in_specs.append(pl.BlockSpec((1, RB, CP), lambda c, r: (c, r, 0)))
        args.append(skip3)

    return pl.pallas_call(
        body,
        grid=(nc, NRB),
        in_specs=in_specs,
        out_specs=pl.BlockSpec((1, RB, CP), lambda c, r: (c, r, 0)),
        out_shape=jax.ShapeDtypeStruct((nc, N, CP), jnp.float32),
    )(*args)


def kernel(x, edge_index, Ws, bs):
    src = edge_index[0]
    dst = edge_index[1]
    pad = EPAD - E
    src_p = jnp.concatenate([src, jnp.zeros((pad,), jnp.int32)])
    dst_p = jnp.concatenate([dst, jnp.full((pad,), N, jnp.int32)])
    src3 = src_p.reshape(NW, NB, B)
    dst4 = jnp.pad(dst_p.reshape(NW, NB, B), ((0, 0), (0, NBS - NB), (0, 0)),
                   constant_values=N)

    zeros_c = jnp.zeros((NPAD // NSUB, CP), jnp.float32)

    src4 = jnp.pad(src3, ((0, 0), (0, NBS - NB), (0, 0)))
    srcs_by_nc = {}
    for d in (128, 640, 320, 160, 80, 40):
        nc = -(-d // CP)
        if nc not in srcs_by_nc:
            srcs_by_nc[nc] = jnp.stack([src4 + c * N for c in range(nc)])

    degp = _sc_scatter(1, srcs_by_nc[1], dst4, jnp.ones((B, CP), jnp.float32),
                       zeros_c, gather=False)
    dis = _tc_dis(degp.reshape(NCORE, N, CP))[:, :1]

    acts = {}
    h3 = x.reshape(1, N, CP)
    for k in range(10):
        W = Ws[k]
        b = bs[k]
        din, dout = W.shape
        nci = h3.shape[0]
        nco = -(-dout // CP)
        Wp = jnp.zeros((nci * CP, nco * CP), jnp.float32)
        Wp = Wp.at[:din, :dout].set(W)
        W3 = Wp.reshape(nci * CP, nco, CP).transpose(1, 0, 2)
        g3 = _tc_matmul(h3, W3, dis)
        gflat = g3.reshape(nco * N, CP)
        P = _sc_scatter(nco, srcs_by_nc[nco], dst4, gflat, zeros_c)
        b3 = jnp.zeros((nco * CP,), jnp.float32).at[:dout].set(b)
        b3 = b3.reshape(nco, 1, CP)
        skip3 = acts.get(9 - k) if 5 <= k <= 8 else None
        h3 = _tc_combine(P, g3, dis, b3, skip3)
        if k <= 3:
            acts[k + 1] = h3

    return h3.reshape(N, CP)


# speedup vs baseline: 1.8312x; 1.8312x over previous
import functools

import jax
import jax.numpy as jnp
from jax import lax
from jax.experimental import pallas as pl
from jax.experimental.pallas import tpu as pltpu
from jax.experimental.pallas import tpu_sc as plsc

N = 10000
E = 320000
NCORE = 2
NSUB = 16
NW = NCORE * NSUB
B = 128
NB = 79
NBS = NB + 1
EPT = NB * B
EPAD = EPT * NW
NPAD = 10112
CP = 128
RB = 2000
NRB = N // RB

_mesh = functools.partial(
    plsc.VectorSubcoreMesh, core_axis_name="c", subcore_axis_name="s"
)


def _sc_scatter(nc, src16, dst4, gflat, zeros_c, gather=True):
    def body(src16_r, dst4_r, g_r, z_r, p_r, srcv16, dstv, gidx, rows, acc,
             semg):
        core = lax.axis_index("c")
        sub = lax.axis_index("s")
        w = core * NSUB + sub
        if gather:
            pltpu.sync_copy(src16_r.at[w], srcv16)
        else:
            pltpu.sync_copy(g_r.at[pl.ds(0, B)], rows)
        pltpu.sync_copy(dst4_r.at[w], dstv)
        for c in range(nc):
            off = jnp.int32(c * N)
            pltpu.sync_copy(z_r, acc.at[pl.ds(sub * (NPAD // NSUB),
                                              NPAD // NSUB)])
            if gather:
                @pl.loop(0, NB)
                def _(j):
                    row = j >> 1
                    col = (j & 1) << 6
                    for t in range(4):
                        v = srcv16[row, pl.ds(col + 16 * t, 16)]
                        gidx[j, pl.ds(32 * t, 16)] = (v & 0xFFFF) + off
                        gidx[j, pl.ds(32 * t + 16, 16)] = (
                            lax.shift_right_logical(v, 16) + off)

            plsc.subcore_barrier()

            @pl.loop(0, NB)
            def _(j):
                if gather:
                    pltpu.async_copy(g_r.at[gidx.at[j]], rows, semg).wait()
                pltpu.sync_copy(rows, acc.at[dstv.at[j]], add=True)

            plsc.subcore_barrier()
            pltpu.sync_copy(
                acc.at[pl.ds(sub * 624, 624)],
                p_r.at[c, core, pl.ds(sub * 624, 624)],
            )

            @pl.when(sub == 15)
            def _():
                pltpu.sync_copy(
                    acc.at[pl.ds(9984, 16)],
                    p_r.at[c, core, pl.ds(9984, 16)],
                )

            plsc.subcore_barrier()

    return pl.kernel(
        body,
        out_type=jax.ShapeDtypeStruct((nc, NCORE, N, CP), jnp.float32),
        mesh=_mesh(),
        scratch_types=[
            pltpu.VMEM((NBS // 2, B), jnp.int32),
            pltpu.VMEM((NBS, B), jnp.int32),
            pltpu.VMEM((NB, B), jnp.int32),
            pltpu.VMEM((B, CP), jnp.float32),
            pltpu.VMEM_SHARED((NPAD, CP), jnp.float32),
            pltpu.SemaphoreType.DMA,
        ],
    )(src16, dst4, gflat, zeros_c)


def _tc_dis(degp):
    def body(d_ref, o_ref):
        o_ref[...] = lax.rsqrt(d_ref[0] + d_ref[1] + 1.0)

    return pl.pallas_call(
        body,
        out_shape=jax.ShapeDtypeStruct((N, CP), jnp.float32),
    )(degp)


def _tc_matmul(h3, W3, dis):
    nci, _, _ = h3.shape
    nco, din_pad, _ = W3.shape

    def body(h_ref, w_ref, dis_ref, o_ref):
        acc = jnp.zeros((RB, CP), jnp.float32)
        for i in range(nci):
            acc += jnp.dot(
                h_ref[i],
                w_ref[0, i * CP : (i + 1) * CP, :],
                preferred_element_type=jnp.float32,
            )
        o_ref[0] = dis_ref[...] * acc

    return pl.pallas_call(
        body,
        grid=(NRB, nco),
        in_specs=[
            pl.BlockSpec((nci, RB, CP), lambda r, c: (0, r, 0)),
            pl.BlockSpec((1, din_pad, CP), lambda r, c: (c, 0, 0)),
            pl.BlockSpec((RB, 1), lambda r, c: (r, 0)),
        ],
        out_specs=pl.BlockSpec((1, RB, CP), lambda r, c: (c, r, 0)),
        out_shape=jax.ShapeDtypeStruct((nco, N, CP), jnp.float32),
    )(h3, W3, dis)


def _tc_fused(P, g3, dis, b3, skip3, W3):
    nci = P.shape[0]
    nco = W3.shape[0]
    has_skip = skip3 is not None

    def body(p_ref, g_ref, dis_ref, b_ref, *rest):
        if has_skip:
            s_ref, w_ref, h_ref, o_ref = rest
        else:
            w_ref, h_ref, o_ref = rest
        d = dis_ref[...]
        accum = jnp.zeros((RB, CP), jnp.float32)
        for i in range(nci):
            v = p_ref[i, 0] + p_ref[i, 1] + g_ref[i]
            v = d * v + b_ref[i]
            if has_skip:
                v += s_ref[i]
            h = jnp.maximum(v, 0.0)
            h_ref[i] = h
            accum += jnp.dot(
                h,
                w_ref[0, i * CP : (i + 1) * CP, :],
                preferred_element_type=jnp.float32,
            )
        o_ref[0] = d * accum

    in_specs = [
        pl.BlockSpec((nci, 2, RB, CP), lambda r, c: (0, 0, r, 0)),
        pl.BlockSpec((nci, RB, CP), lambda r, c: (0, r, 0)),
        pl.BlockSpec((RB, 1), lambda r, c: (r, 0)),
        pl.BlockSpec((nci, 1, CP), lambda r, c: (0, 0, 0)),
    ]
    args = [P, g3, dis, b3]
    if has_skip:
        in_specs.append(pl.BlockSpec((nci, RB, CP), lambda r, c: (0, r, 0)))
        args.append(skip3)
    in_specs.append(pl.BlockSpec((1, nci * CP, CP), lambda r, c: (c, 0, 0)))
    args.append(W3)

    return pl.pallas_call(
        body,
        grid=(NRB, nco),
        in_specs=in_specs,
        out_specs=[
            pl.BlockSpec((nci, RB, CP), lambda r, c: (0, r, 0)),
            pl.BlockSpec((1, RB, CP), lambda r, c: (c, r, 0)),
        ],
        out_shape=[
            jax.ShapeDtypeStruct((nci, N, CP), jnp.float32),
            jax.ShapeDtypeStruct((nco, N, CP), jnp.float32),
        ],
    )(*args)


def _tc_combine(P, g3, dis, b3, skip3):
    nc = P.shape[0]
    has_skip = skip3 is not None

    def body(p_ref, g_ref, dis_ref, b_ref, *rest):
        if has_skip:
            s_ref, o_ref = rest
        else:
            (o_ref,) = rest
        v = p_ref[0, 0] + p_ref[0, 1] + g_ref[0]
        v = dis_ref[...] * v + b_ref[0]
        if has_skip:
            v += s_ref[0]
        o_ref[0] = jnp.maximum(v, 0.0)

    in_specs = [
        pl.BlockSpec((1, 2, RB, CP), lambda c, r: (c, 0, r, 0)),
        pl.BlockSpec((1, RB, CP), lambda c, r: (c, r, 0)),
        pl.BlockSpec((RB, 1), lambda c, r: (r, 0)),
        pl.BlockSpec((1, 1, CP), lambda c, r: (c, 0, 0)),
    ]
    args = [P, g3, dis, b3]
    if has_skip:
        in_specs.append(pl.BlockSpec((1, RB, CP), lambda c, r: (c, r, 0)))
        args.append(skip3)

    return pl.pallas_call(
        body,
        grid=(nc, NRB),
        in_specs=in_specs,
        out_specs=pl.BlockSpec((1, RB, CP), lambda c, r: (c, r, 0)),
        out_shape=jax.ShapeDtypeStruct((nc, N, CP), jnp.float32),
    )(*args)


def kernel(x, edge_index, Ws, bs):
    src = edge_index[0]
    dst = edge_index[1]
    pad = EPAD - E
    src_p = jnp.concatenate([src, jnp.zeros((pad,), jnp.int32)])
    dst_p = jnp.concatenate([dst, jnp.full((pad,), N, jnp.int32)])
    dst4 = jnp.pad(dst_p.reshape(NW, NB, B), ((0, 0), (0, NBS - NB), (0, 0)),
                   constant_values=N)
    sp = src_p.reshape(-1, 2, 16)
    packed = sp[:, 0, :] | (sp[:, 1, :] << 16)
    packed = jnp.pad(packed.reshape(NW, NB, B // 2),
                     ((0, 0), (0, NBS - NB), (0, 0)))
    src16 = packed.reshape(NW, NBS // 2, B)

    zeros_c = jnp.zeros((NPAD // NSUB, CP), jnp.float32)

    degp = _sc_scatter(1, src16, dst4, jnp.ones((B, CP), jnp.float32),
                       zeros_c, gather=False)
    dis = _tc_dis(degp.reshape(NCORE, N, CP))[:, :1]

    def padded_w(W, nci, nco):
        din, dout = W.shape
        Wp = jnp.zeros((nci * CP, nco * CP), jnp.float32)
        Wp = Wp.at[:din, :dout].set(W)
        return Wp.reshape(nci * CP, nco, CP).transpose(1, 0, 2)

    acts = {}
    g3 = _tc_matmul(x.reshape(1, N, CP), padded_w(Ws[0], 1, 5), dis)
    h3 = None
    for k in range(10):
        dout = Ws[k].shape[1]
        nco = -(-dout // CP)
        P = _sc_scatter(nco, src16, dst4, g3.reshape(nco * N, CP), zeros_c)
        b3 = jnp.zeros((nco * CP,), jnp.float32).at[:dout].set(bs[k])
        b3 = b3.reshape(nco, 1, CP)
        skip3 = acts.get(9 - k) if 5 <= k <= 8 else None
        if k < 9:
            nco2 = -(-Ws[k + 1].shape[1] // CP)
            h3, g3 = _tc_fused(P, g3, dis, b3, skip3,
                               padded_w(Ws[k + 1], nco, nco2))
            if k <= 3:
                acts[k + 1] = h3
        else:
            h3 = _tc_combine(P, g3, dis, b3, skip3)

    return h3.reshape(N, CP)
